# Initial kernel scaffold; baseline (speedup 1.0000x reference)
import jax
import jax.numpy as jnp
from jax.experimental import pallas as pl


def kernel(title_ids, token_ids, title_table, token_table):
    raise NotImplementedError("write your pallas kernel here")


# speedup vs baseline: 3.2268x; 3.2268x over previous
import functools

import jax
import jax.numpy as jnp
from jax import lax
from jax.experimental import pallas as pl
from jax.experimental.pallas import tpu as pltpu
from jax.experimental.pallas import tpu_sc as plsc

B = 16384
L = 20
E = 32
NC = 2
NS = 16
NW = NC * NS
BPW = B // NW
CH = 32
NCH = BPW // CH
ROWS = CH * L
IG = 128
TSUB = BPW // IG
KSUB = ROWS // IG

_cached = {}


def _lane_shuffle(v, perm):
    dnums = lax.GatherDimensionNumbers(
        offset_dims=(), collapsed_slice_dims=(0,), start_index_map=(0,))
    return lax.gather(v, perm[:, None], dnums, (1,),
                      mode=lax.GatherScatterMode.PROMISE_IN_BOUNDS)


def _build():
    if "k" in _cached:
        return _cached["k"]

    mesh = plsc.VectorSubcoreMesh(core_axis_name="c", subcore_axis_name="s")

    @functools.partial(
        pl.kernel,
        mesh=mesh,
        out_type=jax.ShapeDtypeStruct((B, 2 * E), jnp.float32),
        compiler_params=pltpu.CompilerParams(use_tc_tiling_on_sc=False),
        scratch_types=[
            pltpu.VMEM((BPW,), jnp.int32),
            pltpu.VMEM((ROWS + 32,), jnp.int32),
            pltpu.VMEM((ROWS, E), jnp.float32),
            pltpu.VMEM((BPW, E), jnp.float32),
            pltpu.VMEM((BPW, 2 * E), jnp.float32),
            pltpu.VMEM((1, E), jnp.float32),
            pltpu.SemaphoreType.DMA,
            pltpu.SemaphoreType.DMA,
        ],
    )
    def movie_sc(title_ids, tok_flat, title_table, token_table, out,
                 tidx_v, cflat_v, rows_v, e1_v, out_v, row0_v, sem_t, sem_r):
        wid = lax.axis_index("s") * NC + lax.axis_index("c")
        base = wid * BPW

        pltpu.sync_copy(title_ids.at[pl.ds(base, BPW)], tidx_v)
        t_copies = [
            pltpu.async_copy(title_table.at[tidx_v.at[pl.ds(k * IG, IG)]],
                             e1_v.at[pl.ds(k * IG, IG)], sem_t)
            for k in range(TSUB)
        ]

        pltpu.sync_copy(token_table.at[pl.ds(0, 1)], row0_v)
        lanes = lax.iota(jnp.int32, 16)
        tailm = lanes < (L - 16)
        perms = [jnp.bitwise_xor(lanes, sh) for sh in (1, 2, 4, 8)]

        def chunk_body(c, carry):
            pltpu.sync_copy(tok_flat.at[pl.ds(base * L + c * ROWS, ROWS)],
                            cflat_v.at[pl.ds(0, ROWS)])
            r_copies = [
                pltpu.async_copy(
                    token_table.at[cflat_v.at[pl.ds(k * IG, IG)]],
                    rows_v.at[pl.ds(k * IG, IG)], sem_r)
                for k in range(KSUB)
            ]
            for cp in r_copies:
                cp.wait()
            r0a = row0_v[0, pl.ds(0, 16)]
            r0b = row0_v[0, pl.ds(16, 16)]

            def title_body(b, carry2):
                r = b * L
                s0 = rows_v[r, pl.ds(0, 16)]
                s1 = rows_v[r, pl.ds(16, 16)]
                for l in range(1, L):
                    s0 = s0 + rows_v[r + l, pl.ds(0, 16)]
                    s1 = s1 + rows_v[r + l, pl.ds(16, 16)]
                one_v = jnp.full((16,), 1.0, jnp.float32)
                zero_v = jnp.full((16,), 0.0, jnp.float32)
                m0 = cflat_v[pl.ds(r, 16)] != 0
                m1 = (cflat_v[pl.ds(r + 16, 16)] != 0) & tailm
                nf = (jnp.where(m0, one_v, zero_v)
                      + jnp.where(m1, one_v, zero_v))
                for p in perms:
                    nf = nf + _lane_shuffle(nf, p)
                pad = jnp.float32(L) - nf
                den = jnp.maximum(nf, 1.0)
                row = c * CH + b
                out_v[row, pl.ds(E, 16)] = (s0 - pad * r0a) / den
                out_v[row, pl.ds(E + 16, 16)] = (s1 - pad * r0b) / den
                return carry2

            return lax.fori_loop(0, CH, title_body, carry)

        lax.fori_loop(0, NCH, chunk_body, 0)

        for cp in t_copies:
            cp.wait()

        def merge_body(b, carry):
            out_v[b, pl.ds(0, 16)] = e1_v[b, pl.ds(0, 16)]
            out_v[b, pl.ds(16, 16)] = e1_v[b, pl.ds(16, 16)]
            return carry

        lax.fori_loop(0, BPW, merge_body, 0)
        pltpu.sync_copy(out_v, out.at[pl.ds(base, BPW)])

    _cached["k"] = movie_sc
    return movie_sc


def kernel(title_ids, token_ids, title_table, token_table):
    title1d = title_ids.astype(jnp.int32).reshape(B)
    tok_flat = token_ids.astype(jnp.int32).reshape(B * L)
    return _build()(title1d, tok_flat, title_table, token_table)
